# baseline (speedup 1.0000x reference)
import functools

import jax
import jax.numpy as jnp
from jax import lax
from jax.experimental import pallas as pl
from jax.experimental.pallas import tpu as pltpu
from jax.experimental.pallas import tpu_sc as plsc

_EPS_BN = 1e-5
_QB = 256
_QC = 32


def _prologue_kernel(xs_ref, ws_ref, ps_ref, x_ref, w_ref, p_ref,
                     o_ref, ab_ref, *, n_rows):
    h = jnp.dot(xs_ref[...], ws_ref[...], preferred_element_type=jnp.float32)
    h = h + ps_ref[0, :][None, :]
    mu = jnp.mean(h, axis=0, keepdims=True)
    var = jnp.mean((h - mu) ** 2, axis=0, keepdims=True)
    h = (h - mu) / jnp.sqrt(var + _EPS_BN)
    h = h * ps_ref[1, :][None, :] + ps_ref[2, :][None, :]
    o_ref[...] = jnp.maximum(h, 0.0)

    x = x_ref[...]
    w = w_ref[...]
    b = p_ref[0, :]
    gamma = p_ref[1, :]
    beta = p_ref[2, :]
    xs = jnp.sum(x, axis=0)
    gram = lax.dot_general(x, x, (((0,), (0,)), ((), ())),
                           preferred_element_type=jnp.float32)
    gw = jnp.dot(gram, w, preferred_element_type=jnp.float32)
    q2 = jnp.sum(w * gw, axis=0)
    s1 = xs @ w
    inv_n = 1.0 / n_rows
    mu = s1 * inv_n + b
    eg2 = (q2 + 2.0 * b * s1) * inv_n + b * b
    var = eg2 - mu * mu
    a = gamma / jnp.sqrt(var + _EPS_BN)
    c = beta - mu * a
    z = jnp.zeros_like(a)
    ab_ref[...] = jnp.stack([a, c] + [z] * 6)


def _knn_kernel(x_ref, pos_ref, post_ref, w_ref, p_ref, ab_ref,
                hrelu_ref, idx_ref, wb_ref, *, n_src):
    g = jnp.dot(x_ref[...], w_ref[...], preferred_element_type=jnp.float32)
    g = g + p_ref[0, :][None, :]
    h = g * ab_ref[0, :][None, :] + ab_ref[1, :][None, :]
    hrelu_ref[...] = jnp.maximum(h, 0.0)

    q = pos_ref[...]
    s = post_ref[...]
    qq = jnp.sum(q * q, axis=1, keepdims=True)
    ss = jnp.sum(s * s, axis=0, keepdims=True)
    q8 = jnp.concatenate([q, jnp.zeros((q.shape[0], 5), jnp.float32)], axis=1)
    cross = jnp.dot(q8.astype(jnp.bfloat16), s.astype(jnp.bfloat16),
                    preferred_element_type=jnp.float32)
    d2 = qq - 2.0 * cross + ss

    iota = jax.lax.broadcasted_iota(jnp.int32, d2.shape, 1).astype(jnp.float32)
    d2m = d2
    idxs, ws = [], []
    den = jnp.zeros((d2.shape[0], 1), dtype=jnp.float32)
    for k in range(3):
        m = jnp.min(d2m, axis=1, keepdims=True)
        cand = jnp.where(d2m == m, iota, jnp.float32(n_src))
        amin = jnp.min(cand, axis=1, keepdims=True)
        wk = 1.0 / jnp.maximum(m, 1e-16)
        idxs.append(amin)
        ws.append(wk)
        den = den + wk
        if k < 2:
            d2m = jnp.where(iota == amin, jnp.float32(jnp.inf), d2m)

    idx_ref[...] = jnp.concatenate(idxs, axis=1).astype(jnp.int32)
    wb_ref[...] = jnp.concatenate(
        [jnp.broadcast_to(w / den, (d2.shape[0], 16)) for w in ws],
        axis=1)


def _sc_interp_kernel(hsub_hbm, hrelu_hbm, idx_hbm, wb_hbm, out_hbm,
                      idx_v0, idx_v1, rows_v0, rows_v1, wb_v, acc_v,
                      sem0, sem1, *, n_q, num_cores):
    wid = lax.axis_index("s") * num_cores + lax.axis_index("c")
    per_w = n_q // (num_cores * 16)
    n_chunks = per_w // _QC

    idx_bufs = (idx_v0, idx_v1)
    row_bufs = (rows_v0, rows_v1)
    sems = (sem0, sem1)

    def start_gather(ci, slot):
        base = wid * per_w + ci * _QC
        pltpu.sync_copy(idx_hbm.at[pl.ds(3 * base, 3 * _QC)], idx_bufs[slot])
        return pltpu.async_copy(hsub_hbm.at[idx_bufs[slot]], row_bufs[slot],
                                sems[slot])

    handles = {0: start_gather(0, 0)}
    for ci in range(n_chunks):
        slot = ci % 2
        if ci + 1 < n_chunks:
            handles[ci + 1] = start_gather(ci + 1, (ci + 1) % 2)
        handles[ci].wait()
        base = wid * per_w + ci * _QC
        pltpu.sync_copy(wb_hbm.at[pl.ds(3 * base, 3 * _QC)], wb_v)
        pltpu.sync_copy(hrelu_hbm.at[pl.ds(base, _QC)], acc_v)
        rows_v = row_bufs[slot]

        def q_body(qi, c2):
            w0 = wb_v[3 * qi, :]
            w1 = wb_v[3 * qi + 1, :]
            w2 = wb_v[3 * qi + 2, :]
            for j in range(16):
                sl = pl.ds(16 * j, 16)
                r0 = rows_v[3 * qi, sl]
                r1 = rows_v[3 * qi + 1, sl]
                r2 = rows_v[3 * qi + 2, sl]
                acc_v[qi, sl] = acc_v[qi, sl] + w0 * r0 + w1 * r1 + w2 * r2
            return c2

        lax.fori_loop(0, _QC, q_body, 0, unroll=False)
        pltpu.sync_copy(acc_v, out_hbm.at[pl.ds(base, _QC)])


def kernel(x, x_sub, pos, pos_sub, W_sub, b_sub, gamma_sub, beta_sub, W, b, gamma, beta):
    n, cout = x.shape
    n_sub = x_sub.shape[0]

    p_sub = jnp.stack([b_sub, gamma_sub, beta_sub] + [jnp.zeros_like(b_sub)] * 5)
    p_main = jnp.stack([b, gamma, beta] + [jnp.zeros_like(b)] * 5)

    h_sub, ab = pl.pallas_call(
        functools.partial(_prologue_kernel, n_rows=n),
        out_shape=[
            jax.ShapeDtypeStruct((n_sub, cout), jnp.float32),
            jax.ShapeDtypeStruct((8, cout), jnp.float32),
        ],
    )(x_sub, W_sub, p_sub, x, W, p_main)

    post = jnp.zeros((8, n_sub), jnp.float32).at[0:3, :].set(pos_sub.T)

    info = plsc.get_sparse_core_info()
    mesh = plsc.VectorSubcoreMesh(core_axis_name="c", subcore_axis_name="s")

    def run_piece(blk0, nbq):
        n_h = nbq * _QB
        hrelu, idx, wb = pl.pallas_call(
            functools.partial(_knn_kernel, n_src=n_sub),
            grid=(nbq,),
            in_specs=[
                pl.BlockSpec((_QB, cout), lambda j: (j + blk0, 0)),
                pl.BlockSpec((_QB, 3), lambda j: (j + blk0, 0)),
                pl.BlockSpec((8, n_sub), lambda j: (0, 0)),
                pl.BlockSpec((cout, cout), lambda j: (0, 0)),
                pl.BlockSpec((8, cout), lambda j: (0, 0)),
                pl.BlockSpec((8, cout), lambda j: (0, 0)),
            ],
            out_specs=[
                pl.BlockSpec((_QB, cout), lambda j: (j, 0)),
                pl.BlockSpec((_QB, 3), lambda j: (j, 0)),
                pl.BlockSpec((_QB, 48), lambda j: (j, 0)),
            ],
            out_shape=[
                jax.ShapeDtypeStruct((n_h, cout), jnp.float32),
                jax.ShapeDtypeStruct((n_h, 3), jnp.int32),
                jax.ShapeDtypeStruct((n_h, 48), jnp.float32),
            ],
        )(x, pos, post, W, p_main, ab)

        idx_flat = idx.reshape(3 * n_h)
        wb_flat = wb.reshape(3 * n_h, 16)
        return pl.kernel(
            functools.partial(_sc_interp_kernel, n_q=n_h,
                              num_cores=info.num_cores),
            mesh=mesh,
            out_type=jax.ShapeDtypeStruct((n_h, cout), jnp.float32),
            scratch_types=[
                pltpu.VMEM((3 * _QC,), jnp.int32),
                pltpu.VMEM((3 * _QC,), jnp.int32),
                pltpu.VMEM((3 * _QC, cout), jnp.float32),
                pltpu.VMEM((3 * _QC, cout), jnp.float32),
                pltpu.VMEM((3 * _QC, 16), jnp.float32),
                pltpu.VMEM((_QC, cout), jnp.float32),
                pltpu.SemaphoreType.DMA,
                pltpu.SemaphoreType.DMA,
            ],
        )(h_sub, hrelu, idx_flat, wb_flat)

    sizes = [20, 20, 16, 8]
    outs, b0 = [], 0
    for nbq in sizes:
        outs.append(run_piece(b0, nbq))
        b0 += nbq
    return jnp.concatenate(outs, axis=0)


# speedup vs baseline: 1.0434x; 1.0434x over previous
import functools

import jax
import jax.numpy as jnp
from jax import lax
from jax.experimental import pallas as pl
from jax.experimental.pallas import tpu as pltpu
from jax.experimental.pallas import tpu_sc as plsc

_EPS_BN = 1e-5
_QB = 256
_QC = 32


def _prologue_kernel(xs_ref, ws_ref, ps_ref, x_ref, w_ref, p_ref,
                     o_ref, ab_ref, *, n_rows):
    h = jnp.dot(xs_ref[...], ws_ref[...], preferred_element_type=jnp.float32)
    h = h + ps_ref[0, :][None, :]
    mu = jnp.mean(h, axis=0, keepdims=True)
    var = jnp.mean((h - mu) ** 2, axis=0, keepdims=True)
    h = (h - mu) / jnp.sqrt(var + _EPS_BN)
    h = h * ps_ref[1, :][None, :] + ps_ref[2, :][None, :]
    o_ref[...] = jnp.maximum(h, 0.0)

    x = x_ref[...]
    w = w_ref[...]
    b = p_ref[0, :]
    gamma = p_ref[1, :]
    beta = p_ref[2, :]
    xs = jnp.sum(x, axis=0)
    gram = lax.dot_general(x, x, (((0,), (0,)), ((), ())),
                           preferred_element_type=jnp.float32)
    gw = jnp.dot(gram, w, preferred_element_type=jnp.float32)
    q2 = jnp.sum(w * gw, axis=0)
    s1 = xs @ w
    inv_n = 1.0 / n_rows
    mu = s1 * inv_n + b
    eg2 = (q2 + 2.0 * b * s1) * inv_n + b * b
    var = eg2 - mu * mu
    a = gamma / jnp.sqrt(var + _EPS_BN)
    c = beta - mu * a
    z = jnp.zeros_like(a)
    ab_ref[...] = jnp.stack([a, c] + [z] * 6)


def _knn_kernel(x_ref, pos_ref, post_ref, w_ref, p_ref, ab_ref,
                hrelu_ref, idx_ref, wb_ref, *, n_src):
    g = jnp.dot(x_ref[...], w_ref[...], preferred_element_type=jnp.float32)
    g = g + p_ref[0, :][None, :]
    h = g * ab_ref[0, :][None, :] + ab_ref[1, :][None, :]
    hrelu_ref[...] = jnp.maximum(h, 0.0)

    q = pos_ref[...]
    s = post_ref[...]
    qq = jnp.sum(q * q, axis=1, keepdims=True)
    ss = jnp.sum(s * s, axis=0, keepdims=True)
    q8 = jnp.concatenate([q, jnp.zeros((q.shape[0], 5), jnp.float32)], axis=1)
    cross = jnp.dot(q8.astype(jnp.bfloat16), s.astype(jnp.bfloat16),
                    preferred_element_type=jnp.float32)
    d2 = qq - 2.0 * cross + ss

    iota = jax.lax.broadcasted_iota(jnp.int32, d2.shape, 1).astype(jnp.float32)
    d2m = d2
    idxs, ws = [], []
    den = jnp.zeros((d2.shape[0], 1), dtype=jnp.float32)
    for k in range(3):
        m = jnp.min(d2m, axis=1, keepdims=True)
        cand = jnp.where(d2m == m, iota, jnp.float32(n_src))
        amin = jnp.min(cand, axis=1, keepdims=True)
        wk = 1.0 / jnp.maximum(m, 1e-16)
        idxs.append(amin)
        ws.append(wk)
        den = den + wk
        if k < 2:
            d2m = jnp.where(iota == amin, jnp.float32(jnp.inf), d2m)

    idx_ref[...] = jnp.concatenate(idxs, axis=1).astype(jnp.int32)
    wb_ref[...] = jnp.concatenate(
        [jnp.broadcast_to(w / den, (d2.shape[0], 16)) for w in ws],
        axis=1)


def _sc_interp_kernel(hsub_hbm, hrelu_hbm, idx_hbm, wb_hbm, out_hbm,
                      idx_v0, idx_v1, rows_v0, rows_v1, wb_v, acc_v,
                      sem0, sem1, *, n_q, num_cores):
    wid = lax.axis_index("s") * num_cores + lax.axis_index("c")
    per_w = n_q // (num_cores * 16)
    n_chunks = per_w // _QC

    idx_bufs = (idx_v0, idx_v1)
    row_bufs = (rows_v0, rows_v1)
    sems = (sem0, sem1)

    def start_gather(ci, slot):
        base = wid * per_w + ci * _QC
        pltpu.sync_copy(idx_hbm.at[pl.ds(3 * base, 3 * _QC)], idx_bufs[slot])
        return pltpu.async_copy(hsub_hbm.at[idx_bufs[slot]], row_bufs[slot],
                                sems[slot])

    handles = {0: start_gather(0, 0)}
    for ci in range(n_chunks):
        slot = ci % 2
        if ci + 1 < n_chunks:
            handles[ci + 1] = start_gather(ci + 1, (ci + 1) % 2)
        handles[ci].wait()
        base = wid * per_w + ci * _QC
        pltpu.sync_copy(wb_hbm.at[pl.ds(3 * base, 3 * _QC)], wb_v)
        pltpu.sync_copy(hrelu_hbm.at[pl.ds(base, _QC)], acc_v)
        rows_v = row_bufs[slot]

        def q_body(qi, c2):
            w0 = wb_v[3 * qi, :]
            w1 = wb_v[3 * qi + 1, :]
            w2 = wb_v[3 * qi + 2, :]
            for j in range(16):
                sl = pl.ds(16 * j, 16)
                r0 = rows_v[3 * qi, sl]
                r1 = rows_v[3 * qi + 1, sl]
                r2 = rows_v[3 * qi + 2, sl]
                acc_v[qi, sl] = acc_v[qi, sl] + w0 * r0 + w1 * r1 + w2 * r2
            return c2

        lax.fori_loop(0, _QC, q_body, 0, unroll=False)
        pltpu.sync_copy(acc_v, out_hbm.at[pl.ds(base, _QC)])


def kernel(x, x_sub, pos, pos_sub, W_sub, b_sub, gamma_sub, beta_sub, W, b, gamma, beta):
    n, cout = x.shape
    n_sub = x_sub.shape[0]

    p_sub = jnp.stack([b_sub, gamma_sub, beta_sub] + [jnp.zeros_like(b_sub)] * 5)
    p_main = jnp.stack([b, gamma, beta] + [jnp.zeros_like(b)] * 5)

    h_sub, ab = pl.pallas_call(
        functools.partial(_prologue_kernel, n_rows=n),
        out_shape=[
            jax.ShapeDtypeStruct((n_sub, cout), jnp.float32),
            jax.ShapeDtypeStruct((8, cout), jnp.float32),
        ],
    )(x_sub, W_sub, p_sub, x, W, p_main)

    post = jnp.zeros((8, n_sub), jnp.float32).at[0:3, :].set(pos_sub.T)

    info = plsc.get_sparse_core_info()
    mesh = plsc.VectorSubcoreMesh(core_axis_name="c", subcore_axis_name="s")

    def run_piece(blk0, nbq):
        n_h = nbq * _QB
        hrelu, idx, wb = pl.pallas_call(
            functools.partial(_knn_kernel, n_src=n_sub),
            grid=(nbq,),
            in_specs=[
                pl.BlockSpec((_QB, cout), lambda j: (j + blk0, 0)),
                pl.BlockSpec((_QB, 3), lambda j: (j + blk0, 0)),
                pl.BlockSpec((8, n_sub), lambda j: (0, 0)),
                pl.BlockSpec((cout, cout), lambda j: (0, 0)),
                pl.BlockSpec((8, cout), lambda j: (0, 0)),
                pl.BlockSpec((8, cout), lambda j: (0, 0)),
            ],
            out_specs=[
                pl.BlockSpec((_QB, cout), lambda j: (j, 0)),
                pl.BlockSpec((_QB, 3), lambda j: (j, 0)),
                pl.BlockSpec((_QB, 48), lambda j: (j, 0)),
            ],
            out_shape=[
                jax.ShapeDtypeStruct((n_h, cout), jnp.float32),
                jax.ShapeDtypeStruct((n_h, 3), jnp.int32),
                jax.ShapeDtypeStruct((n_h, 48), jnp.float32),
            ],
        )(x, pos, post, W, p_main, ab)

        idx_flat = idx.reshape(3 * n_h)
        wb_flat = wb.reshape(3 * n_h, 16)
        return pl.kernel(
            functools.partial(_sc_interp_kernel, n_q=n_h,
                              num_cores=info.num_cores),
            mesh=mesh,
            out_type=jax.ShapeDtypeStruct((n_h, cout), jnp.float32),
            scratch_types=[
                pltpu.VMEM((3 * _QC,), jnp.int32),
                pltpu.VMEM((3 * _QC,), jnp.int32),
                pltpu.VMEM((3 * _QC, cout), jnp.float32),
                pltpu.VMEM((3 * _QC, cout), jnp.float32),
                pltpu.VMEM((3 * _QC, 16), jnp.float32),
                pltpu.VMEM((_QC, cout), jnp.float32),
                pltpu.SemaphoreType.DMA,
                pltpu.SemaphoreType.DMA,
            ],
        )(h_sub, hrelu, idx_flat, wb_flat)

    sizes = [16, 16, 16, 16]
    outs, b0 = [], 0
    for nbq in sizes:
        outs.append(run_piece(b0, nbq))
        b0 += nbq
    return jnp.concatenate(outs, axis=0)


# speedup vs baseline: 1.0741x; 1.0295x over previous
import functools

import jax
import jax.numpy as jnp
from jax import lax
from jax.experimental import pallas as pl
from jax.experimental.pallas import tpu as pltpu
from jax.experimental.pallas import tpu_sc as plsc

_EPS_BN = 1e-5
_QB = 512
_QC = 32


def _prologue_kernel(xs_ref, ws_ref, ps_ref, x_ref, w_ref, p_ref,
                     o_ref, ab_ref, *, n_rows):
    h = jnp.dot(xs_ref[...], ws_ref[...], preferred_element_type=jnp.float32)
    h = h + ps_ref[0, :][None, :]
    mu = jnp.mean(h, axis=0, keepdims=True)
    var = jnp.mean((h - mu) ** 2, axis=0, keepdims=True)
    h = (h - mu) / jnp.sqrt(var + _EPS_BN)
    h = h * ps_ref[1, :][None, :] + ps_ref[2, :][None, :]
    o_ref[...] = jnp.maximum(h, 0.0)

    x = x_ref[...]
    w = w_ref[...]
    b = p_ref[0, :]
    gamma = p_ref[1, :]
    beta = p_ref[2, :]
    xs = jnp.sum(x, axis=0)
    gram = lax.dot_general(x, x, (((0,), (0,)), ((), ())),
                           preferred_element_type=jnp.float32)
    gw = jnp.dot(gram, w, preferred_element_type=jnp.float32)
    q2 = jnp.sum(w * gw, axis=0)
    s1 = xs @ w
    inv_n = 1.0 / n_rows
    mu = s1 * inv_n + b
    eg2 = (q2 + 2.0 * b * s1) * inv_n + b * b
    var = eg2 - mu * mu
    a = gamma / jnp.sqrt(var + _EPS_BN)
    c = beta - mu * a
    z = jnp.zeros_like(a)
    ab_ref[...] = jnp.stack([a, c] + [z] * 6)


def _knn_kernel(x_ref, pos_ref, post_ref, w_ref, p_ref, ab_ref,
                hrelu_ref, idx_ref, wb_ref, *, n_src):
    g = jnp.dot(x_ref[...], w_ref[...], preferred_element_type=jnp.float32)
    g = g + p_ref[0, :][None, :]
    h = g * ab_ref[0, :][None, :] + ab_ref[1, :][None, :]
    hrelu_ref[...] = jnp.maximum(h, 0.0)

    q = pos_ref[...]
    s = post_ref[...]
    qq = jnp.sum(q * q, axis=1, keepdims=True)
    ss = jnp.sum(s * s, axis=0, keepdims=True)
    q8 = jnp.concatenate([q, jnp.zeros((q.shape[0], 5), jnp.float32)], axis=1)
    cross = jnp.dot(q8.astype(jnp.bfloat16), s.astype(jnp.bfloat16),
                    preferred_element_type=jnp.float32)
    d2 = qq - 2.0 * cross + ss

    iota = jax.lax.broadcasted_iota(jnp.int32, d2.shape, 1).astype(jnp.float32)
    d2m = d2
    idxs, ws = [], []
    den = jnp.zeros((d2.shape[0], 1), dtype=jnp.float32)
    for k in range(3):
        m = jnp.min(d2m, axis=1, keepdims=True)
        cand = jnp.where(d2m == m, iota, jnp.float32(n_src))
        amin = jnp.min(cand, axis=1, keepdims=True)
        wk = 1.0 / jnp.maximum(m, 1e-16)
        idxs.append(amin)
        ws.append(wk)
        den = den + wk
        if k < 2:
            d2m = jnp.where(iota == amin, jnp.float32(jnp.inf), d2m)

    idx_ref[...] = jnp.concatenate(idxs, axis=1).astype(jnp.int32)
    wb_ref[...] = jnp.concatenate(
        [jnp.broadcast_to(w / den, (d2.shape[0], 16)) for w in ws],
        axis=1)


def _sc_interp_kernel(hsub_hbm, hrelu_hbm, idx_hbm, wb_hbm, out_hbm,
                      idx_v0, idx_v1, rows_v0, rows_v1, wb_v, acc_v,
                      sem0, sem1, *, n_q, num_cores):
    wid = lax.axis_index("s") * num_cores + lax.axis_index("c")
    per_w = n_q // (num_cores * 16)
    n_chunks = per_w // _QC

    idx_bufs = (idx_v0, idx_v1)
    row_bufs = (rows_v0, rows_v1)
    sems = (sem0, sem1)

    def start_gather(ci, slot):
        base = wid * per_w + ci * _QC
        pltpu.sync_copy(idx_hbm.at[pl.ds(3 * base, 3 * _QC)], idx_bufs[slot])
        return pltpu.async_copy(hsub_hbm.at[idx_bufs[slot]], row_bufs[slot],
                                sems[slot])

    handles = {0: start_gather(0, 0)}
    for ci in range(n_chunks):
        slot = ci % 2
        if ci + 1 < n_chunks:
            handles[ci + 1] = start_gather(ci + 1, (ci + 1) % 2)
        handles[ci].wait()
        base = wid * per_w + ci * _QC
        pltpu.sync_copy(wb_hbm.at[pl.ds(3 * base, 3 * _QC)], wb_v)
        pltpu.sync_copy(hrelu_hbm.at[pl.ds(base, _QC)], acc_v)
        rows_v = row_bufs[slot]

        def q_body(qi, c2):
            w0 = wb_v[3 * qi, :]
            w1 = wb_v[3 * qi + 1, :]
            w2 = wb_v[3 * qi + 2, :]
            for j in range(16):
                sl = pl.ds(16 * j, 16)
                r0 = rows_v[3 * qi, sl]
                r1 = rows_v[3 * qi + 1, sl]
                r2 = rows_v[3 * qi + 2, sl]
                acc_v[qi, sl] = acc_v[qi, sl] + w0 * r0 + w1 * r1 + w2 * r2
            return c2

        lax.fori_loop(0, _QC, q_body, 0, unroll=False)
        pltpu.sync_copy(acc_v, out_hbm.at[pl.ds(base, _QC)])


def kernel(x, x_sub, pos, pos_sub, W_sub, b_sub, gamma_sub, beta_sub, W, b, gamma, beta):
    n, cout = x.shape
    n_sub = x_sub.shape[0]

    p_sub = jnp.stack([b_sub, gamma_sub, beta_sub] + [jnp.zeros_like(b_sub)] * 5)
    p_main = jnp.stack([b, gamma, beta] + [jnp.zeros_like(b)] * 5)

    h_sub, ab = pl.pallas_call(
        functools.partial(_prologue_kernel, n_rows=n),
        out_shape=[
            jax.ShapeDtypeStruct((n_sub, cout), jnp.float32),
            jax.ShapeDtypeStruct((8, cout), jnp.float32),
        ],
    )(x_sub, W_sub, p_sub, x, W, p_main)

    post = jnp.zeros((8, n_sub), jnp.float32).at[0:3, :].set(pos_sub.T)

    info = plsc.get_sparse_core_info()
    mesh = plsc.VectorSubcoreMesh(core_axis_name="c", subcore_axis_name="s")

    def run_piece(blk0, nbq):
        n_h = nbq * _QB
        hrelu, idx, wb = pl.pallas_call(
            functools.partial(_knn_kernel, n_src=n_sub),
            grid=(nbq,),
            in_specs=[
                pl.BlockSpec((_QB, cout), lambda j: (j + blk0, 0)),
                pl.BlockSpec((_QB, 3), lambda j: (j + blk0, 0)),
                pl.BlockSpec((8, n_sub), lambda j: (0, 0)),
                pl.BlockSpec((cout, cout), lambda j: (0, 0)),
                pl.BlockSpec((8, cout), lambda j: (0, 0)),
                pl.BlockSpec((8, cout), lambda j: (0, 0)),
            ],
            out_specs=[
                pl.BlockSpec((_QB, cout), lambda j: (j, 0)),
                pl.BlockSpec((_QB, 3), lambda j: (j, 0)),
                pl.BlockSpec((_QB, 48), lambda j: (j, 0)),
            ],
            out_shape=[
                jax.ShapeDtypeStruct((n_h, cout), jnp.float32),
                jax.ShapeDtypeStruct((n_h, 3), jnp.int32),
                jax.ShapeDtypeStruct((n_h, 48), jnp.float32),
            ],
        )(x, pos, post, W, p_main, ab)

        idx_flat = idx.reshape(3 * n_h)
        wb_flat = wb.reshape(3 * n_h, 16)
        return pl.kernel(
            functools.partial(_sc_interp_kernel, n_q=n_h,
                              num_cores=info.num_cores),
            mesh=mesh,
            out_type=jax.ShapeDtypeStruct((n_h, cout), jnp.float32),
            scratch_types=[
                pltpu.VMEM((3 * _QC,), jnp.int32),
                pltpu.VMEM((3 * _QC,), jnp.int32),
                pltpu.VMEM((3 * _QC, cout), jnp.float32),
                pltpu.VMEM((3 * _QC, cout), jnp.float32),
                pltpu.VMEM((3 * _QC, 16), jnp.float32),
                pltpu.VMEM((_QC, cout), jnp.float32),
                pltpu.SemaphoreType.DMA,
                pltpu.SemaphoreType.DMA,
            ],
        )(h_sub, hrelu, idx_flat, wb_flat)

    sizes = [8, 8, 8, 8]
    outs, b0 = [], 0
    for nbq in sizes:
        outs.append(run_piece(b0, nbq))
        b0 += nbq
    return jnp.concatenate(outs, axis=0)
